# baseline (speedup 1.0000x reference)
import jax
import jax.numpy as jnp
from jax.experimental import pallas as pl
from jax.experimental.pallas import tpu as pltpu

_ROWS = 1536
_COLS = 512
_BLOCK_ROWS = 384


def _qsample_body(t_ref, sac_ref, somac_ref, x_ref, n_ref, o_ref):
    b = pl.program_id(0)
    tt = t_ref[b]
    a = sac_ref[tt]
    s = somac_ref[tt]
    o_ref[...] = a * x_ref[...] + s * n_ref[...]


def kernel(x_start, t, noise, sqrt_alphas_cumprod, sqrt_one_minus_alphas_cumprod):
    B, C, H, W = x_start.shape
    xr = x_start.reshape(B, _ROWS, _COLS)
    nr = noise.reshape(B, _ROWS, _COLS)
    t32 = t.astype(jnp.int32)

    grid = (B, _ROWS // _BLOCK_ROWS)
    spec = pl.BlockSpec((1, _BLOCK_ROWS, _COLS), lambda b, j, *_: (b, j, 0))
    grid_spec = pltpu.PrefetchScalarGridSpec(
        num_scalar_prefetch=3,
        grid=grid,
        in_specs=[spec, spec],
        out_specs=spec,
    )
    out = pl.pallas_call(
        _qsample_body,
        grid_spec=grid_spec,
        out_shape=jax.ShapeDtypeStruct((B, _ROWS, _COLS), jnp.float32),
    )(t32, sqrt_alphas_cumprod, sqrt_one_minus_alphas_cumprod, xr, nr)
    return out.reshape(B, C, H, W)


# speedup vs baseline: 1.3440x; 1.3440x over previous
import jax
import jax.numpy as jnp
from jax.experimental import pallas as pl
from jax.experimental.pallas import tpu as pltpu

_ROWS = 1536
_COLS = 512
_BLOCK_ROWS = 768


def _qsample_body(t_ref, sac_ref, somac_ref, x_ref, n_ref, o_ref):
    b = pl.program_id(0)
    tt = t_ref[b]
    a = sac_ref[tt]
    s = somac_ref[tt]
    o_ref[...] = a * x_ref[...] + s * n_ref[...]


def kernel(x_start, t, noise, sqrt_alphas_cumprod, sqrt_one_minus_alphas_cumprod):
    B, C, H, W = x_start.shape
    xr = x_start.reshape(B, _ROWS, _COLS)
    nr = noise.reshape(B, _ROWS, _COLS)
    t32 = t.astype(jnp.int32)

    grid = (B, _ROWS // _BLOCK_ROWS)
    spec = pl.BlockSpec((1, _BLOCK_ROWS, _COLS), lambda b, j, *_: (b, j, 0))
    grid_spec = pltpu.PrefetchScalarGridSpec(
        num_scalar_prefetch=3,
        grid=grid,
        in_specs=[spec, spec],
        out_specs=spec,
    )
    out = pl.pallas_call(
        _qsample_body,
        grid_spec=grid_spec,
        out_shape=jax.ShapeDtypeStruct((B, _ROWS, _COLS), jnp.float32),
    )(t32, sqrt_alphas_cumprod, sqrt_one_minus_alphas_cumprod, xr, nr)
    return out.reshape(B, C, H, W)


# speedup vs baseline: 1.4290x; 1.0633x over previous
import jax
import jax.numpy as jnp
from jax.experimental import pallas as pl
from jax.experimental.pallas import tpu as pltpu

_ROWS = 1536
_COLS = 512
_BLOCK_ROWS = 1536


def _qsample_body(t_ref, sac_ref, somac_ref, x_ref, n_ref, o_ref):
    b = pl.program_id(0)
    tt = t_ref[b]
    a = sac_ref[tt]
    s = somac_ref[tt]
    o_ref[...] = a * x_ref[...] + s * n_ref[...]


def kernel(x_start, t, noise, sqrt_alphas_cumprod, sqrt_one_minus_alphas_cumprod):
    B, C, H, W = x_start.shape
    xr = x_start.reshape(B, _ROWS, _COLS)
    nr = noise.reshape(B, _ROWS, _COLS)
    t32 = t.astype(jnp.int32)

    grid = (B, _ROWS // _BLOCK_ROWS)
    spec = pl.BlockSpec((1, _BLOCK_ROWS, _COLS), lambda b, j, *_: (b, j, 0))
    grid_spec = pltpu.PrefetchScalarGridSpec(
        num_scalar_prefetch=3,
        grid=grid,
        in_specs=[spec, spec],
        out_specs=spec,
    )
    out = pl.pallas_call(
        _qsample_body,
        grid_spec=grid_spec,
        out_shape=jax.ShapeDtypeStruct((B, _ROWS, _COLS), jnp.float32),
    )(t32, sqrt_alphas_cumprod, sqrt_one_minus_alphas_cumprod, xr, nr)
    return out.reshape(B, C, H, W)
